# baseline (speedup 1.0000x reference)
import functools
import math

import jax
import jax.numpy as jnp
from jax import lax
from jax.experimental import pallas as pl
from jax.experimental.pallas import tpu as pltpu
from jax.experimental.pallas import tpu_sc as plsc

_D = 512
_NC = 2
_NS = 16
_NW = _NC * _NS
_CHUNK = 120
_NBUF = 2
_LANES = 16


def _sc_normalize(descriptors, a, b):
    n, d = descriptors.shape
    oct_total = n // 8
    oct_per = oct_total // _NW
    oct_rem = oct_total % _NW
    n_chunks = (oct_per * 8) // _CHUNK
    assert (oct_per * 8) % _CHUNK == 0 and n_chunks >= _NBUF
    mesh = plsc.VectorSubcoreMesh(core_axis_name="c", subcore_axis_name="s")

    @functools.partial(
        pl.kernel,
        out_type=jax.ShapeDtypeStruct((n, d), jnp.float32),
        mesh=mesh,
        scratch_types=[
            [pltpu.VMEM((_CHUNK, _D), jnp.float32)] * _NBUF,
            pltpu.VMEM((_D,), jnp.float32),
            pltpu.VMEM((_D,), jnp.float32),
            [pltpu.SemaphoreType.DMA] * _NBUF,
            [pltpu.SemaphoreType.DMA] * _NBUF,
        ],
    )
    def k(x_hbm, a_hbm, b_hbm, o_hbm, bufs, a_v, b_v, sis, sos):
        wid = lax.axis_index("s") * _NC + lax.axis_index("c")
        base = (wid * oct_per + jnp.minimum(wid, oct_rem)) * 8

        pltpu.sync_copy(a_hbm, a_v)
        pltpu.sync_copy(b_hbm, b_v)

        def row0(g, rows=_CHUNK):
            return pl.multiple_of(base + g * rows, 8)

        def start_in(g, buf, sem):
            pltpu.async_copy(x_hbm.at[pl.ds(row0(g), _CHUNK)], buf, sem)

        def start_out(g, buf, sem):
            pltpu.async_copy(buf, o_hbm.at[pl.ds(row0(g), _CHUNK)], sem)

        def wait_in(buf, sem):
            pltpu.make_async_copy(x_hbm.at[pl.ds(0, _CHUNK)], buf, sem).wait()

        def wait_out(buf, sem):
            pltpu.make_async_copy(buf, o_hbm.at[pl.ds(0, _CHUNK)], sem).wait()

        def compute(buf, rows):
            unroll = 8
            for j in range(_D // _LANES):
                sl = pl.ds(j * _LANES, _LANES)
                aj = a_v[sl]
                bj = b_v[sl]

                def rbody(r, _, sl=sl, aj=aj, bj=bj):
                    for u in range(unroll):
                        buf[r * unroll + u, sl] = buf[r * unroll + u, sl] * aj + bj
                    return 0

                lax.fori_loop(0, rows // unroll, rbody, 0)

        for s in range(_NBUF):
            start_in(s, bufs[s], sis[s])

        def phase(g, s):
            wait_in(bufs[s], sis[s])
            compute(bufs[s], _CHUNK)
            start_out(g, bufs[s], sos[s])

            @pl.when(g + _NBUF < n_chunks)
            def _():
                wait_out(bufs[s], sos[s])
                start_in(g + _NBUF, bufs[s], sis[s])

        def ring_body(q, _):
            for s in range(_NBUF):
                g = q * _NBUF + s

                @pl.when(g < n_chunks)
                def _(g=g, s=s):
                    phase(g, s)

            return 0

        lax.fori_loop(0, (n_chunks + _NBUF - 1) // _NBUF, ring_body, 0)
        for s in range(_NBUF):
            wait_out(bufs[s], sos[s])

        @pl.when(wid < oct_rem)
        def _():
            tail = pl.multiple_of(base + n_chunks * _CHUNK, 8)
            pltpu.async_copy(
                x_hbm.at[pl.ds(tail, 8)], bufs[0].at[pl.ds(0, 8)], sis[0])
            pltpu.make_async_copy(
                x_hbm.at[pl.ds(0, 8)], bufs[0].at[pl.ds(0, 8)], sis[0]).wait()
            compute(bufs[0], 8)
            pltpu.async_copy(
                bufs[0].at[pl.ds(0, 8)], o_hbm.at[pl.ds(tail, 8)], sos[0])
            pltpu.make_async_copy(
                bufs[0].at[pl.ds(0, 8)], o_hbm.at[pl.ds(0, 8)], sos[0]).wait()

    return k(descriptors, a, b)


_BLOCK_ROWS = 7376


def _norm_body(x_ref, m_hbm, s_hbm, o_ref, tmp_ref, ab_ref, sem):
    d = x_ref.shape[1]

    @pl.when(pl.program_id(0) == 0)
    def _():
        pltpu.make_async_copy(m_hbm, tmp_ref.at[0:1], sem).start()
        pltpu.make_async_copy(s_hbm, tmp_ref.at[1:2], sem).start()
        pltpu.make_async_copy(m_hbm, tmp_ref.at[0:1], sem).wait()
        pltpu.make_async_copy(s_hbm, tmp_ref.at[1:2], sem).wait()
        a = 1.0 / (tmp_ref[1:2, :] * math.sqrt(d))
        ab_ref[0:1, :] = a
        ab_ref[1:2, :] = -tmp_ref[0:1, :] * a

    o_ref[...] = x_ref[...] * ab_ref[0:1, :] + ab_ref[1:2, :]


def _tc_normalize(descriptors, mean, std):
    n, d = descriptors.shape
    grid = pl.cdiv(n, _BLOCK_ROWS)
    return pl.pallas_call(
        _norm_body,
        grid=(grid,),
        in_specs=[
            pl.BlockSpec((_BLOCK_ROWS, d), lambda i: (i, 0)),
            pl.BlockSpec(memory_space=pltpu.HBM),
            pl.BlockSpec(memory_space=pltpu.HBM),
        ],
        out_specs=pl.BlockSpec((_BLOCK_ROWS, d), lambda i: (i, 0)),
        out_shape=jax.ShapeDtypeStruct((n, d), descriptors.dtype),
        scratch_shapes=[
            pltpu.VMEM((2, d), jnp.float32),
            pltpu.VMEM((2, d), jnp.float32),
            pltpu.SemaphoreType.DMA,
        ],
        compiler_params=pltpu.CompilerParams(vmem_limit_bytes=128 * 1024 * 1024),
    )(descriptors, mean, std)


_MC = 4000
_MNB = 4


def _tc_manual(descriptors, a2, b2):
    n, d = descriptors.shape
    n_chunks = pl.cdiv(n, _MC)

    def body(x_hbm, a_hbm, b_hbm, o_hbm, bufs, a_v, b_v, sis, sos):
        pltpu.make_async_copy(a_hbm, a_v, sis[0]).start()
        pltpu.make_async_copy(b_hbm, b_v, sis[0]).start()
        pltpu.make_async_copy(a_hbm, a_v, sis[0]).wait()
        pltpu.make_async_copy(b_hbm, b_v, sis[0]).wait()

        def start_in(g, s):
            pltpu.make_async_copy(
                x_hbm.at[pl.ds(g * _MC, _MC)], bufs[s], sis[s]).start()

        def start_out(g, s):
            pltpu.make_async_copy(
                bufs[s], o_hbm.at[pl.ds(g * _MC, _MC)], sos[s]).start()

        def wait_in(s):
            pltpu.make_async_copy(
                x_hbm.at[pl.ds(0, _MC)], bufs[s], sis[s]).wait()

        def wait_out(s):
            pltpu.make_async_copy(
                bufs[s], o_hbm.at[pl.ds(0, _MC)], sos[s]).wait()

        for s in range(_MNB):
            start_in(s, s)

        def phase(g, s):
            wait_in(s)
            bufs[s][...] = bufs[s][...] * a_v[...] + b_v[...]
            start_out(g, s)

            @pl.when(g + _MNB < n_chunks)
            def _():
                wait_out(s)
                start_in(g + _MNB, s)

        def ring_body(q, _):
            for s in range(_MNB):
                g = q * _MNB + s

                @pl.when(g < n_chunks)
                def _(g=g, s=s):
                    phase(g, s)

            return 0

        lax.fori_loop(0, (n_chunks + _MNB - 1) // _MNB, ring_body, 0)
        for s in range(_MNB):
            wait_out(s)

    return pl.pallas_call(
        body,
        in_specs=[
            pl.BlockSpec(memory_space=pltpu.HBM),
            pl.BlockSpec(memory_space=pltpu.HBM),
            pl.BlockSpec(memory_space=pltpu.HBM),
        ],
        out_specs=pl.BlockSpec(memory_space=pltpu.HBM),
        out_shape=jax.ShapeDtypeStruct((n, d), descriptors.dtype),
        scratch_shapes=[
            [pltpu.VMEM((_MC, d), jnp.float32)] * _MNB,
            pltpu.VMEM((1, d), jnp.float32),
            pltpu.VMEM((1, d), jnp.float32),
            [pltpu.SemaphoreType.DMA] * _MNB,
            [pltpu.SemaphoreType.DMA] * _MNB,
        ],
        compiler_params=pltpu.CompilerParams(vmem_limit_bytes=128 * 1024 * 1024),
    )(descriptors, a2, b2)


def kernel(descriptors, mean, std):
    return _tc_normalize(descriptors, mean, std)


# speedup vs baseline: 1.0165x; 1.0165x over previous
import functools
import math

import jax
import jax.numpy as jnp
from jax import lax
from jax.experimental import pallas as pl
from jax.experimental.pallas import tpu as pltpu
from jax.experimental.pallas import tpu_sc as plsc

_D = 512
_NC = 2
_NS = 16
_NW = _NC * _NS
_CHUNK = 120
_NBUF = 2
_LANES = 16


def _sc_normalize(descriptors, a, b):
    n, d = descriptors.shape
    oct_total = n // 8
    oct_per = oct_total // _NW
    oct_rem = oct_total % _NW
    n_chunks = (oct_per * 8) // _CHUNK
    assert (oct_per * 8) % _CHUNK == 0 and n_chunks >= _NBUF
    mesh = plsc.VectorSubcoreMesh(core_axis_name="c", subcore_axis_name="s")

    @functools.partial(
        pl.kernel,
        out_type=jax.ShapeDtypeStruct((n, d), jnp.float32),
        mesh=mesh,
        scratch_types=[
            [pltpu.VMEM((_CHUNK, _D), jnp.float32)] * _NBUF,
            pltpu.VMEM((_D,), jnp.float32),
            pltpu.VMEM((_D,), jnp.float32),
            [pltpu.SemaphoreType.DMA] * _NBUF,
            [pltpu.SemaphoreType.DMA] * _NBUF,
        ],
    )
    def k(x_hbm, a_hbm, b_hbm, o_hbm, bufs, a_v, b_v, sis, sos):
        wid = lax.axis_index("s") * _NC + lax.axis_index("c")
        base = (wid * oct_per + jnp.minimum(wid, oct_rem)) * 8

        pltpu.sync_copy(a_hbm, a_v)
        pltpu.sync_copy(b_hbm, b_v)

        def row0(g, rows=_CHUNK):
            return pl.multiple_of(base + g * rows, 8)

        def start_in(g, buf, sem):
            pltpu.async_copy(x_hbm.at[pl.ds(row0(g), _CHUNK)], buf, sem)

        def start_out(g, buf, sem):
            pltpu.async_copy(buf, o_hbm.at[pl.ds(row0(g), _CHUNK)], sem)

        def wait_in(buf, sem):
            pltpu.make_async_copy(x_hbm.at[pl.ds(0, _CHUNK)], buf, sem).wait()

        def wait_out(buf, sem):
            pltpu.make_async_copy(buf, o_hbm.at[pl.ds(0, _CHUNK)], sem).wait()

        def compute(buf, rows):
            unroll = 8
            for j in range(_D // _LANES):
                sl = pl.ds(j * _LANES, _LANES)
                aj = a_v[sl]
                bj = b_v[sl]

                def rbody(r, _, sl=sl, aj=aj, bj=bj):
                    for u in range(unroll):
                        buf[r * unroll + u, sl] = buf[r * unroll + u, sl] * aj + bj
                    return 0

                lax.fori_loop(0, rows // unroll, rbody, 0)

        for s in range(_NBUF):
            start_in(s, bufs[s], sis[s])

        def phase(g, s):
            wait_in(bufs[s], sis[s])
            compute(bufs[s], _CHUNK)
            start_out(g, bufs[s], sos[s])

            @pl.when(g + _NBUF < n_chunks)
            def _():
                wait_out(bufs[s], sos[s])
                start_in(g + _NBUF, bufs[s], sis[s])

        def ring_body(q, _):
            for s in range(_NBUF):
                g = q * _NBUF + s

                @pl.when(g < n_chunks)
                def _(g=g, s=s):
                    phase(g, s)

            return 0

        lax.fori_loop(0, (n_chunks + _NBUF - 1) // _NBUF, ring_body, 0)
        for s in range(_NBUF):
            wait_out(bufs[s], sos[s])

        @pl.when(wid < oct_rem)
        def _():
            tail = pl.multiple_of(base + n_chunks * _CHUNK, 8)
            pltpu.async_copy(
                x_hbm.at[pl.ds(tail, 8)], bufs[0].at[pl.ds(0, 8)], sis[0])
            pltpu.make_async_copy(
                x_hbm.at[pl.ds(0, 8)], bufs[0].at[pl.ds(0, 8)], sis[0]).wait()
            compute(bufs[0], 8)
            pltpu.async_copy(
                bufs[0].at[pl.ds(0, 8)], o_hbm.at[pl.ds(tail, 8)], sos[0])
            pltpu.make_async_copy(
                bufs[0].at[pl.ds(0, 8)], o_hbm.at[pl.ds(0, 8)], sos[0]).wait()

    return k(descriptors, a, b)


_BLOCK_ROWS = 7376


def _norm_body(x_ref, m_ref, s_ref, o_ref):
    d = x_ref.shape[1]
    a = 1.0 / (s_ref[...] * math.sqrt(d))
    o_ref[...] = x_ref[...] * a - m_ref[...] * a


def _tc_normalize(descriptors, mean, std):
    n, d = descriptors.shape
    grid = pl.cdiv(n, _BLOCK_ROWS)
    return pl.pallas_call(
        _norm_body,
        grid=(grid,),
        in_specs=[
            pl.BlockSpec((_BLOCK_ROWS, d), lambda i: (i, 0)),
            pl.BlockSpec((1, d), lambda i: (0, 0)),
            pl.BlockSpec((1, d), lambda i: (0, 0)),
        ],
        out_specs=pl.BlockSpec((_BLOCK_ROWS, d), lambda i: (i, 0)),
        out_shape=jax.ShapeDtypeStruct((n, d), descriptors.dtype),
        compiler_params=pltpu.CompilerParams(vmem_limit_bytes=128 * 1024 * 1024),
    )(descriptors, mean, std)


_MC = 4000
_MNB = 4


def _tc_manual(descriptors, a2, b2):
    n, d = descriptors.shape
    n_chunks = pl.cdiv(n, _MC)

    def body(x_hbm, a_hbm, b_hbm, o_hbm, bufs, a_v, b_v, sis, sos):
        pltpu.make_async_copy(a_hbm, a_v, sis[0]).start()
        pltpu.make_async_copy(b_hbm, b_v, sis[0]).start()
        pltpu.make_async_copy(a_hbm, a_v, sis[0]).wait()
        pltpu.make_async_copy(b_hbm, b_v, sis[0]).wait()

        def start_in(g, s):
            pltpu.make_async_copy(
                x_hbm.at[pl.ds(g * _MC, _MC)], bufs[s], sis[s]).start()

        def start_out(g, s):
            pltpu.make_async_copy(
                bufs[s], o_hbm.at[pl.ds(g * _MC, _MC)], sos[s]).start()

        def wait_in(s):
            pltpu.make_async_copy(
                x_hbm.at[pl.ds(0, _MC)], bufs[s], sis[s]).wait()

        def wait_out(s):
            pltpu.make_async_copy(
                bufs[s], o_hbm.at[pl.ds(0, _MC)], sos[s]).wait()

        for s in range(_MNB):
            start_in(s, s)

        def phase(g, s):
            wait_in(s)
            bufs[s][...] = bufs[s][...] * a_v[...] + b_v[...]
            start_out(g, s)

            @pl.when(g + _MNB < n_chunks)
            def _():
                wait_out(s)
                start_in(g + _MNB, s)

        def ring_body(q, _):
            for s in range(_MNB):
                g = q * _MNB + s

                @pl.when(g < n_chunks)
                def _(g=g, s=s):
                    phase(g, s)

            return 0

        lax.fori_loop(0, (n_chunks + _MNB - 1) // _MNB, ring_body, 0)
        for s in range(_MNB):
            wait_out(s)

    return pl.pallas_call(
        body,
        in_specs=[
            pl.BlockSpec(memory_space=pltpu.HBM),
            pl.BlockSpec(memory_space=pltpu.HBM),
            pl.BlockSpec(memory_space=pltpu.HBM),
        ],
        out_specs=pl.BlockSpec(memory_space=pltpu.HBM),
        out_shape=jax.ShapeDtypeStruct((n, d), descriptors.dtype),
        scratch_shapes=[
            [pltpu.VMEM((_MC, d), jnp.float32)] * _MNB,
            pltpu.VMEM((1, d), jnp.float32),
            pltpu.VMEM((1, d), jnp.float32),
            [pltpu.SemaphoreType.DMA] * _MNB,
            [pltpu.SemaphoreType.DMA] * _MNB,
        ],
        compiler_params=pltpu.CompilerParams(vmem_limit_bytes=128 * 1024 * 1024),
    )(descriptors, a2, b2)


def kernel(descriptors, mean, std):
    return _tc_normalize(descriptors, mean, std)
